# baseline (speedup 1.0000x reference)
import functools

import jax
import jax.numpy as jnp
from jax import lax
from jax.experimental import pallas as pl
from jax.experimental.pallas import tpu as pltpu
from jax.experimental.pallas import tpu_sc as plsc

MAX_NEIGHBORS = 10
PAD_VALUE = -1
EMBED_D = 128

_NC = 2
_NS = 16
_NW = _NC * _NS
_EPR = 4


def _sc_gather_sums(idx16_d, idx_c, idx_g, w_c, w_d, w_g, ids,
                    cell_features, drug_features, gene_features):
  B = ids.shape[0]
  Dc = cell_features.shape[1]
  Dd = drug_features.shape[1]
  Dg = gene_features.shape[1]
  assert B % _NW == 0
  epw = B // _NW
  ipw = epw * MAX_NEIGHBORS

  mesh = plsc.VectorSubcoreMesh(core_axis_name="c", subcore_axis_name="s")

  @functools.partial(
      pl.kernel,
      out_type=[
          jax.ShapeDtypeStruct((B, Dc), jnp.float32),
          jax.ShapeDtypeStruct((B, Dd), jnp.float32),
          jax.ShapeDtypeStruct((B, Dg), jnp.float32),
          jax.ShapeDtypeStruct((B, Dd), jnp.float32),
      ],
      mesh=mesh,
      scratch_types=[
          pltpu.VMEM((epw * 12,), jnp.int32),
          pltpu.VMEM((ipw,), jnp.int32),
          pltpu.VMEM((epw * 16,), jnp.float32),
          pltpu.SemaphoreType.DMA,
          pltpu.SemaphoreType.DMA,
      ],
  )
  def k(idx16_d_h, idx_c_h, idx_g_h, w_c_h, w_d_h, w_g_h, ids_h,
        cell_h, drug_h, gene_h,
        sums_c_h, sums_d_h, sums_g_h, self_h,
        idx16_v, idx_v, w_v, sem_a, sem_b):
    wid = lax.axis_index("s") * _NC + lax.axis_index("c")
    base_e = wid * epw

    def accum_one(rows_v, row0, out_v, w_v, e_loc, e_out, D):
      nch = D // 16
      wvec = w_v[pl.ds(pl.multiple_of(e_loc * 16, 16), 16)]
      ws = [wvec[j] for j in range(MAX_NEIGHBORS)]

      def chunk(c, carry):
        off = pl.multiple_of(c * 16, 16)
        acc0 = rows_v[row0 + 0, pl.ds(off, 16)] * ws[0]
        acc1 = rows_v[row0 + 1, pl.ds(off, 16)] * ws[1]
        for j in range(2, MAX_NEIGHBORS, 2):
          acc0 = acc0 + rows_v[row0 + j, pl.ds(off, 16)] * ws[j]
          acc1 = acc1 + rows_v[row0 + j + 1, pl.ds(off, 16)] * ws[j + 1]
        out_v[e_out, pl.ds(off, 16)] = acc0 + acc1
        return carry

      lax.fori_loop(0, nch, chunk, 0)

    def drug_phase(rows_a, rows_b, out_v):
      npairs = epw // 2
      pltpu.sync_copy(idx16_d_h.at[pl.ds(base_e * 12, epw * 12)], idx16_v)
      pltpu.sync_copy(w_d_h.at[pl.ds(base_e * 16, epw * 16)], w_v)

      def start(buf, sem, p):
        off = pl.multiple_of(p * 24, 8)
        pltpu.make_async_copy(
            drug_h.at[idx16_v.at[pl.ds(off, 24)]], buf, sem).start()

      def wait(buf, sem):
        pltpu.make_async_copy(drug_h.at[pl.ds(0, 24)], buf, sem).wait()

      def accum_pair(rows_v, p):
        e0 = p * 2
        accum_one(rows_v, 0, out_v, w_v, e0, e0 % 8, Dd)
        accum_one(rows_v, 12, out_v, w_v, e0 + 1, (e0 + 1) % 8, Dd)

      start(rows_a, sem_a, 0)

      def body(g, carry):
        p0 = g * 2
        start(rows_b, sem_b, p0 + 1)
        wait(rows_a, sem_a)
        accum_pair(rows_a, p0)

        @pl.when(g + 1 < npairs // 2)
        def _():
          start(rows_a, sem_a, p0 + 2)

        wait(rows_b, sem_b)
        accum_pair(rows_b, p0 + 1)

        @pl.when(g % 2 == 1)
        def _():
          blk = g // 2
          pltpu.sync_copy(out_v,
                          sums_d_h.at[pl.ds(base_e + blk * 8, 8)])

        return carry

      lax.fori_loop(0, npairs // 2, body, 0)

    def small_phase(idx_h, w_h, tbl_h, out_h, rows_a, rows_b, out_v, D, epg):
      nit = epw // epg
      pltpu.sync_copy(idx_h.at[pl.ds(base_e * MAX_NEIGHBORS, ipw)], idx_v)
      pltpu.sync_copy(w_h.at[pl.ds(base_e * 16, epw * 16)], w_v)
      npr = epg * MAX_NEIGHBORS

      def start(buf, sem, it):
        off = pl.multiple_of(it * npr, 8)
        pltpu.make_async_copy(
            tbl_h.at[idx_v.at[pl.ds(off, npr)]], buf, sem).start()

      def wait(buf, sem):
        pltpu.make_async_copy(tbl_h.at[pl.ds(0, npr)], buf, sem).wait()

      def accum_it(rows_v, it):
        for kk in range(epg):
          accum_one(rows_v, kk * MAX_NEIGHBORS,
                    out_v, w_v, it * epg + kk, it * epg + kk, D)

      start(rows_a, sem_a, 0)

      def body(g, carry):
        it0 = g * 2
        start(rows_b, sem_b, it0 + 1)
        wait(rows_a, sem_a)
        accum_it(rows_a, it0)

        @pl.when(g + 1 < nit // 2)
        def _():
          start(rows_a, sem_a, it0 + 2)

        wait(rows_b, sem_b)
        accum_it(rows_b, it0 + 1)
        return carry

      lax.fori_loop(0, nit // 2, body, 0)
      pltpu.sync_copy(out_v, out_h.at[pl.ds(base_e, epw)])

    def self_phase(rows_a, rows_b, rpg):
      nit = epw // rpg
      pltpu.sync_copy(ids_h.at[pl.ds(base_e, epw)],
                      idx_v.at[pl.ds(0, epw)])

      def start(buf, sem, it):
        off = pl.multiple_of(it * rpg, 8)
        pltpu.make_async_copy(
            drug_h.at[idx_v.at[pl.ds(off, rpg)]], buf, sem).start()

      def wait(buf, sem):
        pltpu.make_async_copy(drug_h.at[pl.ds(0, rpg)], buf, sem).wait()

      def store(buf, it):
        pltpu.sync_copy(buf, self_h.at[pl.ds(base_e + it * rpg, rpg)])

      start(rows_a, sem_a, 0)

      def body(g, carry):
        it0 = g * 2
        start(rows_b, sem_b, it0 + 1)
        wait(rows_a, sem_a)
        store(rows_a, it0)

        @pl.when(g + 1 < nit // 2)
        def _():
          start(rows_a, sem_a, it0 + 2)

        wait(rows_b, sem_b)
        store(rows_b, it0 + 1)
        return carry

      lax.fori_loop(0, nit // 2, body, 0)

    pl.run_scoped(
        lambda ra, rb, ov: small_phase(idx_c_h, w_c_h, cell_h, sums_c_h,
                                       ra, rb, ov, Dc, 8),
        pltpu.VMEM((8 * MAX_NEIGHBORS, Dc), jnp.float32),
        pltpu.VMEM((8 * MAX_NEIGHBORS, Dc), jnp.float32),
        pltpu.VMEM((epw, Dc), jnp.float32),
    )
    pl.run_scoped(
        lambda ra, rb, ov: small_phase(idx_g_h, w_g_h, gene_h, sums_g_h,
                                       ra, rb, ov, Dg, 8),
        pltpu.VMEM((8 * MAX_NEIGHBORS, Dg), jnp.float32),
        pltpu.VMEM((8 * MAX_NEIGHBORS, Dg), jnp.float32),
        pltpu.VMEM((epw, Dg), jnp.float32),
    )
    pl.run_scoped(
        drug_phase,
        pltpu.VMEM((24, Dd), jnp.float32),
        pltpu.VMEM((24, Dd), jnp.float32),
        pltpu.VMEM((8, Dd), jnp.float32),
    )
    pl.run_scoped(
        lambda ra, rb: self_phase(ra, rb, 8),
        pltpu.VMEM((8, Dd), jnp.float32),
        pltpu.VMEM((8, Dd), jnp.float32),
    )

  return k(idx16_d, idx_c, idx_g, w_c, w_d, w_g, ids,
           cell_features, drug_features, gene_features)


def _tc_mlp(self_rows, sums_c, sums_d, sums_g, cnt_c, cnt_d, cnt_g,
            W_cell, b_cell, W_drug, b_drug, W_gene, b_gene,
            W_l1, b_l1, W_l2, b_l2):
  B = self_rows.shape[0]
  BLK = 256
  grid = (B // BLK,)
  f32 = jnp.float32
  inv_m = 1.0 / MAX_NEIGHBORS

  def dot(a, b):
    return lax.dot_general(a, b, (((1,), (0,)), ((), ())),
                           preferred_element_type=f32)

  def body(self_r, sc_r, sd_r, sg_r, cc_r, cd_r, cg_r,
           Wc_r, bc_r, Wd_r, bd_r, Wg_r, bg_r,
           Wl1_r, bl1_r, Wl2_r, bl2_r, out_r):
    h = dot(self_r[...], Wd_r[...]) + bd_r[...]
    agg_c = (dot(sc_r[...], Wc_r[...]) + cc_r[...] * bc_r[...]) * inv_m
    agg_d = (dot(sd_r[...], Wd_r[...]) + cd_r[...] * bd_r[...]) * inv_m
    agg_g = (dot(sg_r[...], Wg_r[...]) + cg_r[...] * bg_r[...]) * inv_m
    for Wl_r, bl_r in ((Wl1_r, bl1_r), (Wl2_r, bl2_r)):
      Wl = Wl_r[...]
      pre = (dot(h, Wl[0:EMBED_D]) + dot(agg_c, Wl[EMBED_D:2 * EMBED_D])
             + dot(agg_d, Wl[2 * EMBED_D:3 * EMBED_D])
             + dot(agg_g, Wl[3 * EMBED_D:4 * EMBED_D]) + bl_r[...])
      h = jnp.maximum(pre, 0.0)
    out_r[...] = h

  def rows_spec(d):
    return pl.BlockSpec((BLK, d), lambda i: (i, 0))

  def full_spec(shape):
    return pl.BlockSpec(shape, lambda i: tuple(0 for _ in shape))

  Dc, Dd, Dg = sums_c.shape[1], sums_d.shape[1], sums_g.shape[1]
  b2 = lambda v: v.reshape(1, EMBED_D)
  c2 = lambda v: v.reshape(B, 1)
  return pl.pallas_call(
      body,
      grid=grid,
      in_specs=[
          rows_spec(Dd), rows_spec(Dc), rows_spec(Dd), rows_spec(Dg),
          rows_spec(1), rows_spec(1), rows_spec(1),
          full_spec((Dc, EMBED_D)), full_spec((1, EMBED_D)),
          full_spec((Dd, EMBED_D)), full_spec((1, EMBED_D)),
          full_spec((Dg, EMBED_D)), full_spec((1, EMBED_D)),
          full_spec((4 * EMBED_D, EMBED_D)), full_spec((1, EMBED_D)),
          full_spec((4 * EMBED_D, EMBED_D)), full_spec((1, EMBED_D)),
      ],
      out_specs=rows_spec(EMBED_D),
      out_shape=jax.ShapeDtypeStruct((B, EMBED_D), f32),
  )(self_rows, sums_c, sums_d, sums_g, c2(cnt_c), c2(cnt_d), c2(cnt_g),
    W_cell, b2(b_cell), W_drug, b2(b_drug), W_gene, b2(b_gene),
    W_l1, b2(b_l1), W_l2, b2(b_l2))


def kernel(id_batch, neigh_cell, neigh_drug, neigh_gene,
           cell_features, drug_features, gene_features,
           W_cell, b_cell, W_drug, b_drug, W_gene, b_gene,
           W_l1, b_l1, W_l2, b_l2):
  def prep(neigh):
    mask = neigh != PAD_VALUE
    safe = jnp.where(mask, neigh, 0).astype(jnp.int32)
    idx = safe.reshape(-1)
    idx12 = jnp.pad(safe, ((0, 0), (0, 12 - MAX_NEIGHBORS))).reshape(-1)
    w = jnp.pad(mask.astype(jnp.float32),
                ((0, 0), (0, 16 - MAX_NEIGHBORS))).reshape(-1)
    cnt = mask.sum(axis=1).astype(jnp.float32)
    return idx, idx12, w, cnt

  idx_c, _, w_c, cnt_c = prep(neigh_cell)
  _, idx16_d, w_d, cnt_d = prep(neigh_drug)
  idx_g, _, w_g, cnt_g = prep(neigh_gene)

  sums_c, sums_d, sums_g, self_rows = _sc_gather_sums(
      idx16_d, idx_c, idx_g, w_c, w_d, w_g, id_batch.astype(jnp.int32),
      cell_features, drug_features, gene_features)

  return _tc_mlp(self_rows, sums_c, sums_d, sums_g, cnt_c, cnt_d, cnt_g,
                 W_cell, b_cell, W_drug, b_drug, W_gene, b_gene,
                 W_l1, b_l1, W_l2, b_l2)
